# Initial kernel scaffold; baseline (speedup 1.0000x reference)
import jax
import jax.numpy as jnp
from jax.experimental import pallas as pl


def kernel(x, pos, lw0, lb0, lw1, lb1, s1w0, s1b0, s1w1, s1b1, s2w0, s2b0, s2w1, s2b1):
    raise NotImplementedError("write your pallas kernel here")


# speedup vs baseline: 1.0037x; 1.0037x over previous
import jax
import jax.numpy as jnp
from jax.experimental import pallas as pl
from jax.experimental.pallas import tpu as pltpu

K_NEIGH = 64


def _local_mlp_body(pos_ref, w0_ref, b0_ref, w1_ref, b1_ref, out_ref):
    p = pos_ref[...]
    h = jnp.tanh(p @ w0_ref[...] + b0_ref[...])
    out_ref[...] = jnp.tanh(h @ w1_ref[...] + b1_ref[...])


def _local_mlp(pos, lw0, lb0, lw1, lb1):
    B, N, _ = pos.shape
    pos2 = pos.reshape(B * N, 3)
    out = pl.pallas_call(
        _local_mlp_body,
        grid=(B,),
        in_specs=[
            pl.BlockSpec((N, 3), lambda b: (b, 0)),
            pl.BlockSpec((3, 64), lambda b: (0, 0)),
            pl.BlockSpec((64,), lambda b: (0,)),
            pl.BlockSpec((64, 128), lambda b: (0, 0)),
            pl.BlockSpec((128,), lambda b: (0,)),
        ],
        out_specs=pl.BlockSpec((N, 128), lambda b: (b, 0)),
        out_shape=jax.ShapeDtypeStruct((B * N, 128), jnp.float32),
    )(pos2, lw0, lb0, lw1, lb1)
    return out.reshape(B, N, 128)


def _fps(pos, n_sample):
    pos = jax.lax.stop_gradient(pos)
    idxs = jnp.zeros((n_sample,), dtype=jnp.int32)
    dists = jnp.sum((pos - pos[0]) ** 2, axis=-1)

    def body(i, state):
        dists, idxs = state
        nxt = jnp.argmax(dists).astype(jnp.int32)
        idxs = idxs.at[i].set(nxt)
        d = jnp.sum((pos - pos[nxt]) ** 2, axis=-1)
        dists = jnp.minimum(dists, d)
        return (dists, idxs)

    dists, idxs = jax.lax.fori_loop(1, n_sample, body, (dists, idxs))
    return idxs


def _sa(x, pos, ratio, r, W0, b0, W1, b1):
    N = pos.shape[0]
    n = int(round(N * ratio))
    idx = _fps(pos, n)
    pos_c = pos[idx]
    d2 = jnp.sum((pos_c[:, None, :] - pos[None, :, :]) ** 2, axis=-1)
    d2_sg = jax.lax.stop_gradient(d2)
    neg = jnp.where(d2_sg <= r * r, -d2_sg, -jnp.inf)
    vals, nidx = jax.lax.top_k(neg, K_NEIGH)
    valid = jnp.isfinite(vals)
    xj = x[nidx]
    pj = pos[nidx]
    rel = pj - pos_c[:, None, :]
    msg = jnp.concatenate([xj, rel], axis=-1)
    h = jnp.tanh(msg @ W0 + b0)
    h = jnp.tanh(h @ W1 + b1)
    h = jnp.where(valid[:, :, None], h, -jnp.inf)
    out = jnp.max(h, axis=1)
    out = jnp.where(valid.any(axis=1)[:, None], out, 0.0)
    return out, pos_c


def kernel(x, pos, lw0, lb0, lw1, lb1, s1w0, s1b0, s1w1, s1b1, s2w0, s2b0, s2w1, s2b1):
    local = _local_mlp(pos, lw0, lb0, lw1, lb1)
    gin = jnp.concatenate([pos, x], axis=-1)

    def per_sample(gx, gp):
        h1, p1 = _sa(gx, gp, 0.25, 0.2, s1w0, s1b0, s1w1, s1b1)
        h2, _ = _sa(h1, p1, 0.25, 0.4, s2w0, s2b0, s2w1, s2b1)
        return h2

    gfeat = jax.vmap(per_sample)(gin, pos)
    gfeat = jnp.max(gfeat, axis=1, keepdims=True)
    return (local, gfeat)


# speedup vs baseline: 1.2530x; 1.2484x over previous
import jax
import jax.numpy as jnp
from jax.experimental import pallas as pl
from jax.experimental.pallas import tpu as pltpu

K_NEIGH = 64


def _local_mlp_body(pos_ref, w0_ref, b0_ref, w1_ref, b1_ref, out_ref):
    p = pos_ref[...]
    h = jnp.tanh(p @ w0_ref[...] + b0_ref[...])
    out_ref[...] = jnp.tanh(h @ w1_ref[...] + b1_ref[...])


def _local_mlp(pos, lw0, lb0, lw1, lb1):
    B, N, _ = pos.shape
    pos2 = pos.reshape(B * N, 3)
    out = pl.pallas_call(
        _local_mlp_body,
        grid=(B,),
        in_specs=[
            pl.BlockSpec((N, 3), lambda b: (b, 0)),
            pl.BlockSpec((3, 64), lambda b: (0, 0)),
            pl.BlockSpec((64,), lambda b: (0,)),
            pl.BlockSpec((64, 128), lambda b: (0, 0)),
            pl.BlockSpec((128,), lambda b: (0,)),
        ],
        out_specs=pl.BlockSpec((N, 128), lambda b: (b, 0)),
        out_shape=jax.ShapeDtypeStruct((B * N, 128), jnp.float32),
    )(pos2, lw0, lb0, lw1, lb1)
    return out.reshape(B, N, 128)


def _fps_body(n_sample, n_rows, soa_ref, idx_ref, dists_ref):
    px = soa_ref[0, 0]
    py = soa_ref[0, 1]
    pz = soa_ref[0, 2]
    cx = soa_ref[0, 0, 0, 0]
    cy = soa_ref[0, 1, 0, 0]
    cz = soa_ref[0, 2, 0, 0]
    dx = px - cx
    dy = py - cy
    dz = pz - cz
    dists_ref[...] = dx * dx + dy * dy + dz * dz
    idx_ref[0, pl.ds(0, 1), :] = jnp.zeros((1, 1), jnp.int32)

    flat_iota = (
        jax.lax.broadcasted_iota(jnp.int32, (n_rows, 128), 0) * 128
        + jax.lax.broadcasted_iota(jnp.int32, (n_rows, 128), 1)
    )
    lane_iota = jax.lax.broadcasted_iota(jnp.int32, (1, 128), 1)
    n_pts = n_rows * 128

    def body(i, _):
        dists = dists_ref[...]
        m = jnp.max(dists)
        nxt = jnp.min(jnp.where(dists == m, flat_iota, n_pts))
        idx_ref[0, pl.ds(i, 1), :] = jnp.full((1, 1), nxt, jnp.int32)
        r = nxt // 128
        col = nxt % 128
        lane_mask = lane_iota == col
        cx = jnp.sum(jnp.where(lane_mask, soa_ref[0, 0, pl.ds(r, 1), :], 0.0))
        cy = jnp.sum(jnp.where(lane_mask, soa_ref[0, 1, pl.ds(r, 1), :], 0.0))
        cz = jnp.sum(jnp.where(lane_mask, soa_ref[0, 2, pl.ds(r, 1), :], 0.0))
        dx = px - cx
        dy = py - cy
        dz = pz - cz
        d = dx * dx + dy * dy + dz * dz
        dists_ref[...] = jnp.minimum(dists, d)
        return 0

    jax.lax.fori_loop(1, n_sample, body, 0)


def _fps_batched(pos, n_sample):
    B, N, _ = pos.shape
    n_rows = N // 128
    soa = pos.transpose(0, 2, 1).reshape(B, 3, n_rows, 128)
    import functools
    idx = pl.pallas_call(
        functools.partial(_fps_body, n_sample, n_rows),
        grid=(B,),
        in_specs=[pl.BlockSpec((1, 3, n_rows, 128), lambda b: (b, 0, 0, 0))],
        out_specs=pl.BlockSpec((1, n_sample, 1), lambda b: (b, 0, 0)),
        out_shape=jax.ShapeDtypeStruct((B, n_sample, 1), jnp.int32),
        scratch_shapes=[pltpu.VMEM((n_rows, 128), jnp.float32)],
    )(soa)
    return idx.reshape(B, n_sample)


def _sa_batched(x, pos, ratio, r, W0, b0, W1, b1):
    B, N, _ = pos.shape
    n = int(round(N * ratio))
    idx = _fps_batched(pos, n)
    pos_c = jnp.take_along_axis(pos, idx[..., None], axis=1)
    d2 = jnp.sum((pos_c[:, :, None, :] - pos[:, None, :, :]) ** 2, axis=-1)
    neg = jnp.where(d2 <= r * r, -d2, -jnp.inf)
    vals, nidx = jax.lax.top_k(neg, K_NEIGH)
    valid = jnp.isfinite(vals)
    xj = jax.vmap(lambda xb, ib: xb[ib])(x, nidx)
    pj = jax.vmap(lambda pb, ib: pb[ib])(pos, nidx)
    rel = pj - pos_c[:, :, None, :]
    msg = jnp.concatenate([xj, rel], axis=-1)
    h = jnp.tanh(msg @ W0 + b0)
    h = jnp.tanh(h @ W1 + b1)
    h = jnp.where(valid[..., None], h, -jnp.inf)
    out = jnp.max(h, axis=2)
    out = jnp.where(valid.any(axis=2)[..., None], out, 0.0)
    return out, pos_c


def kernel(x, pos, lw0, lb0, lw1, lb1, s1w0, s1b0, s1w1, s1b1, s2w0, s2b0, s2w1, s2b1):
    local = _local_mlp(pos, lw0, lb0, lw1, lb1)
    gin = jnp.concatenate([pos, x], axis=-1)
    h1, p1 = _sa_batched(gin, pos, 0.25, 0.2, s1w0, s1b0, s1w1, s1b1)
    h2, _ = _sa_batched(h1, p1, 0.25, 0.4, s2w0, s2b0, s2w1, s2b1)
    gfeat = jnp.max(h2, axis=1, keepdims=True)
    return (local, gfeat)


# speedup vs baseline: 1.7512x; 1.3977x over previous
import functools

import jax
import jax.numpy as jnp
from jax.experimental import pallas as pl
from jax.experimental.pallas import tpu as pltpu

K_NEIGH = 64


def _local_mlp_body(pos_ref, w0_ref, b0_ref, w1_ref, b1_ref, out_ref):
    p = pos_ref[...]
    h = jnp.tanh(p @ w0_ref[...] + b0_ref[...])
    out_ref[...] = jnp.tanh(h @ w1_ref[...] + b1_ref[...])


def _local_mlp(pos, lw0, lb0, lw1, lb1):
    B, N, _ = pos.shape
    pos2 = pos.reshape(B * N, 3)
    out = pl.pallas_call(
        _local_mlp_body,
        grid=(B,),
        in_specs=[
            pl.BlockSpec((N, 3), lambda b: (b, 0)),
            pl.BlockSpec((3, 64), lambda b: (0, 0)),
            pl.BlockSpec((64,), lambda b: (0,)),
            pl.BlockSpec((64, 128), lambda b: (0, 0)),
            pl.BlockSpec((128,), lambda b: (0,)),
        ],
        out_specs=pl.BlockSpec((N, 128), lambda b: (b, 0)),
        out_shape=jax.ShapeDtypeStruct((B * N, 128), jnp.float32),
    )(pos2, lw0, lb0, lw1, lb1)
    return out.reshape(B, N, 128)


def _ap_body(feat_ref, pos_ref, w0_ref, wp_ref, b0_ref, a_ref, p_ref):
    f = feat_ref[...]
    p = pos_ref[...]
    wp = wp_ref[...]
    a_ref[...] = f @ w0_ref[...] + p @ wp + b0_ref[...]
    p_ref[...] = p @ wp


def _precompute_ap(feat, pos, W0, b0):
    B, N, df = feat.shape
    dh = W0.shape[1]
    wf = W0[:df]
    wp = W0[df:]
    f2 = feat.reshape(B * N, df)
    p2 = pos.reshape(B * N, 3)
    a, p = pl.pallas_call(
        _ap_body,
        grid=(B,),
        in_specs=[
            pl.BlockSpec((N, df), lambda b: (b, 0)),
            pl.BlockSpec((N, 3), lambda b: (b, 0)),
            pl.BlockSpec((df, dh), lambda b: (0, 0)),
            pl.BlockSpec((3, dh), lambda b: (0, 0)),
            pl.BlockSpec((dh,), lambda b: (0,)),
        ],
        out_specs=[
            pl.BlockSpec((N, dh), lambda b: (b, 0)),
            pl.BlockSpec((N, dh), lambda b: (b, 0)),
        ],
        out_shape=[
            jax.ShapeDtypeStruct((B * N, dh), jnp.float32),
            jax.ShapeDtypeStruct((B * N, dh), jnp.float32),
        ],
    )(f2, p2, wf, wp, b0)
    return a.reshape(B, N, dh), p.reshape(B, N, dh)


def _fps_body(n_sample, n_rows, soa_ref, idx_ref, dists_ref):
    px = soa_ref[0, 0]
    py = soa_ref[0, 1]
    pz = soa_ref[0, 2]
    cx = soa_ref[0, 0, 0, 0]
    cy = soa_ref[0, 1, 0, 0]
    cz = soa_ref[0, 2, 0, 0]
    dx = px - cx
    dy = py - cy
    dz = pz - cz
    dists_ref[...] = dx * dx + dy * dy + dz * dz
    idx_ref[0, pl.ds(0, 1), :] = jnp.zeros((1, 1), jnp.int32)

    flat_iota = (
        jax.lax.broadcasted_iota(jnp.int32, (n_rows, 128), 0) * 128
        + jax.lax.broadcasted_iota(jnp.int32, (n_rows, 128), 1)
    )
    lane_iota = jax.lax.broadcasted_iota(jnp.int32, (1, 128), 1)
    n_pts = n_rows * 128

    def body(i, _):
        dists = dists_ref[...]
        m = jnp.max(dists)
        nxt = jnp.min(jnp.where(dists == m, flat_iota, n_pts))
        idx_ref[0, pl.ds(i, 1), :] = jnp.full((1, 1), nxt, jnp.int32)
        r = nxt // 128
        col = nxt % 128
        lane_mask = lane_iota == col
        cx = jnp.sum(jnp.where(lane_mask, soa_ref[0, 0, pl.ds(r, 1), :], 0.0))
        cy = jnp.sum(jnp.where(lane_mask, soa_ref[0, 1, pl.ds(r, 1), :], 0.0))
        cz = jnp.sum(jnp.where(lane_mask, soa_ref[0, 2, pl.ds(r, 1), :], 0.0))
        dx = px - cx
        dy = py - cy
        dz = pz - cz
        d = dx * dx + dy * dy + dz * dz
        dists_ref[...] = jnp.minimum(dists, d)
        return 0

    jax.lax.fori_loop(1, n_sample, body, 0)


def _fps_batched(pos, n_sample):
    B, N, _ = pos.shape
    n_rows = N // 128
    soa = pos.transpose(0, 2, 1).reshape(B, 3, n_rows, 128)
    idx = pl.pallas_call(
        functools.partial(_fps_body, n_sample, n_rows),
        grid=(B,),
        in_specs=[pl.BlockSpec((1, 3, n_rows, 128), lambda b: (b, 0, 0, 0))],
        out_specs=pl.BlockSpec((1, n_sample, 1), lambda b: (b, 0, 0)),
        out_shape=jax.ShapeDtypeStruct((B, n_sample, 1), jnp.int32),
        scratch_shapes=[pltpu.VMEM((n_rows, 128), jnp.float32)],
    )(soa)
    return idx.reshape(B, n_sample)


def _sa_mlp_body(CB, K, reduce_all, ag_ref, pc_ref, cnt_ref, madd_ref, w1_ref,
                 b1_ref, out_ref):
    j = pl.program_id(1)
    ag = ag_ref[...]
    pc = pc_ref[...]
    dh = ag.shape[-1]
    pre = ag.reshape(CB, K, dh) - pc[:, None, :]
    h = jnp.tanh(pre).reshape(CB * K, dh)
    h2 = jnp.tanh(h @ w1_ref[...] + b1_ref[...])
    do = h2.shape[-1]
    h2 = h2 + madd_ref[...]
    h3 = h2.reshape(CB, K, do)
    cnt = cnt_ref[...]
    mx = jnp.max(h3, axis=1)
    mx = jnp.where(cnt > 0, mx, 0.0)
    if reduce_all:
        blockmax = jnp.max(mx, axis=0, keepdims=True).reshape(1, 1, do)
        @pl.when(j == 0)
        def _():
            out_ref[...] = blockmax
        @pl.when(j > 0)
        def _():
            out_ref[...] = jnp.maximum(out_ref[...], blockmax)
    else:
        out_ref[...] = mx


def _sa_mlp(ag, pc, cnt, madd, W1, b1, CB, reduce_all):
    B, nK, DH = ag.shape
    n = pc.shape[1]
    K = nK // n
    DO = W1.shape[1]
    nb = n // CB
    ag2 = ag.reshape(B * nK, DH)
    pc2 = pc.reshape(B * n, DH)
    cnt2 = cnt.reshape(B * n, 1)
    madd2 = madd.reshape(B * nK, 1)
    if reduce_all:
        out_specs = pl.BlockSpec((1, 1, DO), lambda b, j: (b, 0, 0))
        out_shape = jax.ShapeDtypeStruct((B, 1, DO), jnp.float32)
    else:
        out_specs = pl.BlockSpec((CB, DO), lambda b, j: (b * nb + j, 0))
        out_shape = jax.ShapeDtypeStruct((B * n, DO), jnp.float32)
    out = pl.pallas_call(
        functools.partial(_sa_mlp_body, CB, K, reduce_all),
        grid=(B, nb),
        in_specs=[
            pl.BlockSpec((CB * K, DH), lambda b, j: (b * nb + j, 0)),
            pl.BlockSpec((CB, DH), lambda b, j: (b * nb + j, 0)),
            pl.BlockSpec((CB, 1), lambda b, j: (b * nb + j, 0)),
            pl.BlockSpec((CB * K, 1), lambda b, j: (b * nb + j, 0)),
            pl.BlockSpec((DH, DO), lambda b, j: (0, 0)),
            pl.BlockSpec((DO,), lambda b, j: (0,)),
        ],
        out_specs=out_specs,
        out_shape=out_shape,
    )(ag2, pc2, cnt2, madd2, W1, b1)
    if reduce_all:
        return out
    return out.reshape(B, n, DO)


def _sa_layer(feat, pos, ratio, r, W0, b0, W1, b1, CB, reduce_all):
    B, N, df = feat.shape
    n = int(round(N * ratio))
    A, P = _precompute_ap(feat, pos, W0, b0)
    idx = _fps_batched(pos, n)
    pos_c = jnp.take_along_axis(pos, idx[..., None], axis=1)
    d2 = jnp.sum((pos_c[:, :, None, :] - pos[:, None, :, :]) ** 2, axis=-1)
    neg = jnp.where(d2 <= r * r, -d2, -jnp.inf)
    vals, nidx = jax.lax.top_k(neg, K_NEIGH)
    finite = jnp.isfinite(vals)
    cnt = jnp.sum(finite, axis=-1).astype(jnp.float32)
    madd = jnp.where(finite, 0.0, -jnp.inf)
    nidx = jnp.where(finite, nidx, 0)
    ag = jax.vmap(lambda Ab, ib: Ab[ib])(A, nidx.reshape(B, n * K_NEIGH))
    pcg = jnp.take_along_axis(P, idx[..., None], axis=1)
    h = _sa_mlp(ag, pcg, cnt, madd, W1, b1, CB, reduce_all)
    return h, pos_c


def kernel(x, pos, lw0, lb0, lw1, lb1, s1w0, s1b0, s1w1, s1b1, s2w0, s2b0, s2w1, s2b1):
    local = _local_mlp(pos, lw0, lb0, lw1, lb1)
    gin = jnp.concatenate([pos, x], axis=-1)
    h1, p1 = _sa_layer(gin, pos, 0.25, 0.2, s1w0, s1b0, s1w1, s1b1,
                       CB=128, reduce_all=False)
    gfeat, _ = _sa_layer(h1, p1, 0.25, 0.4, s2w0, s2b0, s2w1, s2b1,
                         CB=64, reduce_all=True)
    return (local, gfeat)
